# Initial kernel scaffold; baseline (speedup 1.0000x reference)
import jax
import jax.numpy as jnp
from jax.experimental import pallas as pl


def kernel(x, edge_index, W1, b1, g1, be1, W2, b2, g2, be2, W3, b3, g3, be3, Wfc, bfc):
    raise NotImplementedError("write your pallas kernel here")


# speedup vs baseline: 10.4224x; 10.4224x over previous
import functools

import jax
import jax.numpy as jnp
from jax import lax
from jax.experimental import pallas as pl
from jax.experimental.pallas import tpu as pltpu
from jax.experimental.pallas import tpu_sc as plsc

_EPS = 1e-5
_CH = 80
_DEGW = 128


def _sc_mesh():
    return plsc.VectorSubcoreMesh(core_axis_name="c", subcore_axis_name="s")


def _row_split(N, NS):
    rows_a = (N // NS) & ~7
    tail = N - NS * rows_a
    return rows_a, tail


def _make_deg1_kernel(N, E):
    NC, NS = 2, 16
    per_core = E // NC
    per_tile = per_core // NS
    n_chunks = per_tile // _CH
    rows_a, tail = _row_split(N, NS)

    @functools.partial(
        pl.kernel,
        out_type=jax.ShapeDtypeStruct((NC, N), jnp.float32),
        mesh=_sc_mesh(),
        scratch_types=[
            pltpu.VMEM((_CH,), jnp.int32),
            pltpu.VMEM((_CH,), jnp.float32),
            pltpu.VMEM_SHARED((N,), jnp.float32),
        ],
    )
    def deg_kernel(dst_hbm, ones_hbm, zeros_hbm, out_hbm, dst_v, ones_v, acc_sh):
        c = lax.axis_index("c")
        s = lax.axis_index("s")
        r0 = s * rows_a
        pltpu.sync_copy(zeros_hbm.at[pl.ds(r0, rows_a)],
                        acc_sh.at[pl.ds(r0, rows_a)])

        @pl.when(s == NS - 1)
        def _():
            pltpu.sync_copy(zeros_hbm.at[pl.ds(NS * rows_a, tail)],
                            acc_sh.at[pl.ds(NS * rows_a, tail)])

        pltpu.sync_copy(ones_hbm, ones_v)
        plsc.subcore_barrier()

        base = c * per_core + s * per_tile

        def body(i, carry):
            off = pl.multiple_of(base + i * _CH, 8)
            pltpu.sync_copy(dst_hbm.at[pl.ds(off, _CH)], dst_v)
            pltpu.sync_copy(ones_v, acc_sh.at[dst_v], add=True)
            return carry

        lax.fori_loop(0, n_chunks, body, 0)
        plsc.subcore_barrier()
        pltpu.sync_copy(acc_sh.at[pl.ds(r0, rows_a)],
                        out_hbm.at[c, pl.ds(r0, rows_a)])

        @pl.when(s == NS - 1)
        def _():
            pltpu.sync_copy(acc_sh.at[pl.ds(NS * rows_a, tail)],
                            out_hbm.at[c, pl.ds(NS * rows_a, tail)])

    return deg_kernel


def _make_deg_kernel(N, E):
    NC, NS = 2, 16
    per_core = E // NC
    per_tile = per_core // NS
    n_chunks = per_tile // _CH
    rows_a, tail = _row_split(N, NS)

    @functools.partial(
        pl.kernel,
        out_type=jax.ShapeDtypeStruct((NC, N, _DEGW), jnp.float32),
        mesh=_sc_mesh(),
        scratch_types=[
            pltpu.VMEM((_CH,), jnp.int32),
            pltpu.VMEM((_CH, _DEGW), jnp.float32),
            pltpu.VMEM_SHARED((N, _DEGW), jnp.float32),
        ],
    )
    def deg_kernel(dst_hbm, ones_hbm, zeros_hbm, out_hbm, dst_v, ones_v, acc_sh):
        c = lax.axis_index("c")
        s = lax.axis_index("s")
        r0 = s * rows_a
        pltpu.sync_copy(zeros_hbm.at[pl.ds(r0, rows_a)],
                        acc_sh.at[pl.ds(r0, rows_a)])

        @pl.when(s == NS - 1)
        def _():
            pltpu.sync_copy(zeros_hbm.at[pl.ds(NS * rows_a, tail)],
                            acc_sh.at[pl.ds(NS * rows_a, tail)])

        pltpu.sync_copy(ones_hbm, ones_v)
        plsc.subcore_barrier()

        base = c * per_core + s * per_tile

        def body(i, carry):
            off = pl.multiple_of(base + i * _CH, 8)
            pltpu.sync_copy(dst_hbm.at[pl.ds(off, _CH)], dst_v)
            pltpu.sync_copy(ones_v, acc_sh.at[dst_v], add=True)
            return carry

        lax.fori_loop(0, n_chunks, body, 0)
        plsc.subcore_barrier()
        pltpu.sync_copy(acc_sh.at[pl.ds(r0, rows_a)],
                        out_hbm.at[c, pl.ds(r0, rows_a)])

        @pl.when(s == NS - 1)
        def _():
            pltpu.sync_copy(acc_sh.at[pl.ds(NS * rows_a, tail)],
                            out_hbm.at[c, pl.ds(NS * rows_a, tail)])

    return deg_kernel


def _make_scatter_kernel(N, E, H):
    NC, NS = 2, 16
    per_core = E // NC
    per_tile = per_core // NS
    n_chunks = per_tile // _CH
    rows_a, tail = _row_split(N, NS)

    @functools.partial(
        pl.kernel,
        out_type=jax.ShapeDtypeStruct((NC, N, H), jnp.float32),
        mesh=_sc_mesh(),
        scratch_types=[
            pltpu.VMEM((_CH,), jnp.int32),
            pltpu.VMEM((_CH,), jnp.int32),
            pltpu.VMEM((_CH, H), jnp.float32),
            pltpu.VMEM_SHARED((N, H), jnp.float32),
            pltpu.SemaphoreType.DMA,
        ],
    )
    def scatter_kernel(hp_hbm, src_hbm, dst_hbm, zeros_hbm, out_hbm,
                       src_v, dst_v, rows_v, acc_sh, sem):
        c = lax.axis_index("c")
        s = lax.axis_index("s")
        r0 = s * rows_a
        pltpu.sync_copy(zeros_hbm.at[pl.ds(r0, rows_a)],
                        acc_sh.at[pl.ds(r0, rows_a)])

        @pl.when(s == NS - 1)
        def _():
            pltpu.sync_copy(zeros_hbm.at[pl.ds(NS * rows_a, tail)],
                            acc_sh.at[pl.ds(NS * rows_a, tail)])

        plsc.subcore_barrier()

        base = c * per_core + s * per_tile

        def body(i, carry):
            off = pl.multiple_of(base + i * _CH, 8)
            pltpu.sync_copy(src_hbm.at[pl.ds(off, _CH)], src_v)
            pltpu.sync_copy(dst_hbm.at[pl.ds(off, _CH)], dst_v)
            pltpu.async_copy(hp_hbm.at[src_v], rows_v, sem).wait()
            pltpu.sync_copy(rows_v, acc_sh.at[dst_v], add=True)
            return carry

        lax.fori_loop(0, n_chunks, body, 0)
        plsc.subcore_barrier()
        pltpu.sync_copy(acc_sh.at[pl.ds(r0, rows_a)],
                        out_hbm.at[c, pl.ds(r0, rows_a)])

        @pl.when(s == NS - 1)
        def _():
            pltpu.sync_copy(acc_sh.at[pl.ds(NS * rows_a, tail)],
                            out_hbm.at[c, pl.ds(NS * rows_a, tail)])

    return scatter_kernel


def _tc0_body(x_ref, w_ref, d0_ref, d1_ref, hp_ref, dinv_ref):
    deg = d0_ref[:, 0:1] + d1_ref[:, 0:1] + 1.0
    dinv = lax.rsqrt(deg)
    dinv_ref[...] = dinv
    pre = jnp.dot(x_ref[...], w_ref[...], preferred_element_type=jnp.float32)
    hp_ref[...] = pre * dinv


def _tc_mid_body(s0_ref, s1_ref, hp_ref, dinv_ref, b_ref, g_ref, be_ref,
                 w_ref, hn_ref, hnp_ref):
    dinv = dinv_ref[...]
    z = dinv * (s0_ref[...] + s1_ref[...] + hp_ref[...]) + b_ref[...]
    mu = jnp.mean(z, axis=0, keepdims=True)
    var = jnp.mean((z - mu) ** 2, axis=0, keepdims=True)
    hn = jnp.maximum((z - mu) * lax.rsqrt(var + _EPS) * g_ref[...] + be_ref[...],
                     0.0)
    hn_ref[...] = hn
    hnp_ref[...] = jnp.dot(hn, w_ref[...],
                           preferred_element_type=jnp.float32) * dinv


def _tc_final_body(s0_ref, s1_ref, hp_ref, dinv_ref, b_ref, g_ref, be_ref,
                   res_ref, wfc_ref, bfc_ref, out_ref):
    dinv = dinv_ref[...]
    z = dinv * (s0_ref[...] + s1_ref[...] + hp_ref[...]) + b_ref[...]
    mu = jnp.mean(z, axis=0, keepdims=True)
    var = jnp.mean((z - mu) ** 2, axis=0, keepdims=True)
    h = jnp.maximum((z - mu) * lax.rsqrt(var + _EPS) * g_ref[...] + be_ref[...],
                    0.0)
    h = h + res_ref[...]
    logits = jnp.dot(h, wfc_ref[...], preferred_element_type=jnp.float32)
    out_ref[...] = jax.nn.sigmoid(logits + bfc_ref[...]) * 10.0


def _tc_call(body, out_shapes, *args):
    return pl.pallas_call(
        body,
        out_shape=out_shapes,
    )(*args)


def kernel(x, edge_index, W1, b1, g1, be1, W2, b2, g2, be2, W3, b3, g3, be3,
           Wfc, bfc):
    N, D = x.shape
    H = W1.shape[1]
    E = edge_index.shape[1]

    src = edge_index[0]
    dst = edge_index[1]

    zerosNH = jnp.zeros((N, H), jnp.float32)
    zerosND = jnp.zeros((N, _DEGW), jnp.float32)
    ones_rows = jnp.ones((_CH, _DEGW), jnp.float32)

    b1r, g1r, be1r = b1.reshape(1, H), g1.reshape(1, H), be1.reshape(1, H)
    b2r, g2r, be2r = b2.reshape(1, H), g2.reshape(1, H), be2.reshape(1, H)
    b3r, g3r, be3r = b3.reshape(1, H), g3.reshape(1, H), be3.reshape(1, H)
    bfcr = bfc.reshape(1, 1)

    deg_k = _make_deg_kernel(N, E)
    scat_k = _make_scatter_kernel(N, E, H)

    degp = deg_k(dst, ones_rows, zerosND)

    h1p, dinv = _tc_call(
        _tc0_body,
        (jax.ShapeDtypeStruct((N, H), jnp.float32),
         jax.ShapeDtypeStruct((N, 1), jnp.float32)),
        x, W1, degp[0], degp[1])

    S1 = scat_k(h1p, src, dst, zerosNH)
    h1, h2p = _tc_call(
        _tc_mid_body,
        (jax.ShapeDtypeStruct((N, H), jnp.float32),
         jax.ShapeDtypeStruct((N, H), jnp.float32)),
        S1[0], S1[1], h1p, dinv, b1r, g1r, be1r, W2)

    S2 = scat_k(h2p, src, dst, zerosNH)
    _, h3p = _tc_call(
        _tc_mid_body,
        (jax.ShapeDtypeStruct((N, H), jnp.float32),
         jax.ShapeDtypeStruct((N, H), jnp.float32)),
        S2[0], S2[1], h2p, dinv, b2r, g2r, be2r, W3)

    S3 = scat_k(h3p, src, dst, zerosNH)
    out = _tc_call(
        _tc_final_body,
        jax.ShapeDtypeStruct((N, 1), jnp.float32),
        S3[0], S3[1], h3p, dinv, b3r, g3r, be3r, h1, Wfc, bfcr)

    return out


# speedup vs baseline: 15.3360x; 1.4714x over previous
import functools

import jax
import jax.numpy as jnp
from jax import lax
from jax.experimental import pallas as pl
from jax.experimental.pallas import tpu as pltpu
from jax.experimental.pallas import tpu_sc as plsc

_EPS = 1e-5
_CH = 80
_DEGW = 128


def _sc_mesh():
    return plsc.VectorSubcoreMesh(core_axis_name="c", subcore_axis_name="s")


def _row_split(N, NS):
    rows_a = (N // NS) & ~7
    tail = N - NS * rows_a
    return rows_a, tail


def _make_deg1_kernel(N, E):
    NC, NS = 2, 16
    per_core = E // NC
    per_tile = per_core // NS
    n_chunks = per_tile // _CH
    rows_a, tail = _row_split(N, NS)

    @functools.partial(
        pl.kernel,
        out_type=jax.ShapeDtypeStruct((NC, N), jnp.float32),
        mesh=_sc_mesh(),
        scratch_types=[
            pltpu.VMEM((_CH,), jnp.int32),
            pltpu.VMEM((_CH,), jnp.float32),
            pltpu.VMEM_SHARED((N,), jnp.float32),
        ],
    )
    def deg_kernel(dst_hbm, ones_hbm, zeros_hbm, out_hbm, dst_v, ones_v, acc_sh):
        c = lax.axis_index("c")
        s = lax.axis_index("s")
        r0 = s * rows_a
        pltpu.sync_copy(zeros_hbm.at[pl.ds(r0, rows_a)],
                        acc_sh.at[pl.ds(r0, rows_a)])

        @pl.when(s == NS - 1)
        def _():
            pltpu.sync_copy(zeros_hbm.at[pl.ds(NS * rows_a, tail)],
                            acc_sh.at[pl.ds(NS * rows_a, tail)])

        pltpu.sync_copy(ones_hbm, ones_v)
        plsc.subcore_barrier()

        base = c * per_core + s * per_tile

        def body(i, carry):
            off = pl.multiple_of(base + i * _CH, 8)
            pltpu.sync_copy(dst_hbm.at[pl.ds(off, _CH)], dst_v)
            pltpu.sync_copy(ones_v, acc_sh.at[dst_v], add=True)
            return carry

        lax.fori_loop(0, n_chunks, body, 0)
        plsc.subcore_barrier()
        pltpu.sync_copy(acc_sh.at[pl.ds(r0, rows_a)],
                        out_hbm.at[c, pl.ds(r0, rows_a)])

        @pl.when(s == NS - 1)
        def _():
            pltpu.sync_copy(acc_sh.at[pl.ds(NS * rows_a, tail)],
                            out_hbm.at[c, pl.ds(NS * rows_a, tail)])

    return deg_kernel


def _make_deg_kernel(N, E):
    NC, NS = 2, 16
    per_core = E // NC
    per_tile = per_core // NS
    n_chunks = per_tile // _CH
    rows_a, tail = _row_split(N, NS)

    @functools.partial(
        pl.kernel,
        out_type=jax.ShapeDtypeStruct((NC, N, _DEGW), jnp.float32),
        mesh=_sc_mesh(),
        scratch_types=[
            pltpu.VMEM((_CH,), jnp.int32),
            pltpu.VMEM((_CH, _DEGW), jnp.float32),
            pltpu.VMEM_SHARED((N, _DEGW), jnp.float32),
        ],
    )
    def deg_kernel(dst_hbm, ones_hbm, zeros_hbm, out_hbm, dst_v, ones_v, acc_sh):
        c = lax.axis_index("c")
        s = lax.axis_index("s")
        r0 = s * rows_a
        pltpu.sync_copy(zeros_hbm.at[pl.ds(r0, rows_a)],
                        acc_sh.at[pl.ds(r0, rows_a)])

        @pl.when(s == NS - 1)
        def _():
            pltpu.sync_copy(zeros_hbm.at[pl.ds(NS * rows_a, tail)],
                            acc_sh.at[pl.ds(NS * rows_a, tail)])

        pltpu.sync_copy(ones_hbm, ones_v)
        plsc.subcore_barrier()

        base = c * per_core + s * per_tile

        def body(i, carry):
            off = pl.multiple_of(base + i * _CH, 8)
            pltpu.sync_copy(dst_hbm.at[pl.ds(off, _CH)], dst_v)
            pltpu.sync_copy(ones_v, acc_sh.at[dst_v], add=True)
            return carry

        lax.fori_loop(0, n_chunks, body, 0)
        plsc.subcore_barrier()
        pltpu.sync_copy(acc_sh.at[pl.ds(r0, rows_a)],
                        out_hbm.at[c, pl.ds(r0, rows_a)])

        @pl.when(s == NS - 1)
        def _():
            pltpu.sync_copy(acc_sh.at[pl.ds(NS * rows_a, tail)],
                            out_hbm.at[c, pl.ds(NS * rows_a, tail)])

    return deg_kernel


def _make_scatter_kernel(N, E, H):
    NC, NS = 2, 16
    per_core = E // NC
    per_tile = per_core // NS
    n_chunks = per_tile // _CH
    rows_a, tail = _row_split(N, NS)

    @functools.partial(
        pl.kernel,
        out_type=jax.ShapeDtypeStruct((NC, N, H), jnp.float32),
        mesh=_sc_mesh(),
        scratch_types=[
            pltpu.VMEM((_CH,), jnp.int32),
            pltpu.VMEM((_CH,), jnp.int32),
            pltpu.VMEM((_CH,), jnp.int32),
            pltpu.VMEM((_CH,), jnp.int32),
            pltpu.VMEM((_CH, H), jnp.float32),
            pltpu.VMEM((_CH, H), jnp.float32),
            pltpu.VMEM_SHARED((N, H), jnp.float32),
            pltpu.SemaphoreType.DMA,
            pltpu.SemaphoreType.DMA,
        ],
    )
    def scatter_kernel(hp_hbm, src_hbm, dst_hbm, zeros_hbm, out_hbm,
                       src_v0, src_v1, dst_v0, dst_v1, rows_v0, rows_v1,
                       acc_sh, sem0, sem1):
        c = lax.axis_index("c")
        s = lax.axis_index("s")
        r0 = s * rows_a
        pltpu.sync_copy(zeros_hbm.at[pl.ds(r0, rows_a)],
                        acc_sh.at[pl.ds(r0, rows_a)])

        @pl.when(s == NS - 1)
        def _():
            pltpu.sync_copy(zeros_hbm.at[pl.ds(NS * rows_a, tail)],
                            acc_sh.at[pl.ds(NS * rows_a, tail)])

        plsc.subcore_barrier()

        base = c * per_core + s * per_tile
        bufs = ((src_v0, dst_v0, rows_v0, sem0), (src_v1, dst_v1, rows_v1, sem1))

        def load_and_gather(i, b):
            src_v, dst_v, rows_v, sem = bufs[b]
            off = pl.multiple_of(base + i * _CH, 8)
            pltpu.sync_copy(src_hbm.at[pl.ds(off, _CH)], src_v)
            pltpu.sync_copy(dst_hbm.at[pl.ds(off, _CH)], dst_v)
            pltpu.async_copy(hp_hbm.at[src_v], rows_v, sem)

        load_and_gather(0, 0)
        load_and_gather(1, 1)

        def step(i, b):
            src_v, dst_v, rows_v, sem = bufs[b]
            pltpu.make_async_copy(hp_hbm.at[src_v], rows_v, sem).wait()
            pltpu.sync_copy(rows_v, acc_sh.at[dst_v], add=True)

            @pl.when(i + 2 < n_chunks)
            def _():
                load_and_gather(i + 2, b)

        def body(i, carry):
            @pl.when(lax.rem(i, 2) == 0)
            def _():
                step(i, 0)

            @pl.when(lax.rem(i, 2) == 1)
            def _():
                step(i, 1)

            return carry

        lax.fori_loop(0, n_chunks, body, 0)
        plsc.subcore_barrier()
        pltpu.sync_copy(acc_sh.at[pl.ds(r0, rows_a)],
                        out_hbm.at[c, pl.ds(r0, rows_a)])

        @pl.when(s == NS - 1)
        def _():
            pltpu.sync_copy(acc_sh.at[pl.ds(NS * rows_a, tail)],
                            out_hbm.at[c, pl.ds(NS * rows_a, tail)])

    return scatter_kernel


def _tc0_body(x_ref, w_ref, d0_ref, d1_ref, hp_ref, dinv_ref):
    deg = d0_ref[:, 0:1] + d1_ref[:, 0:1] + 1.0
    dinv = lax.rsqrt(deg)
    dinv_ref[...] = dinv
    pre = jnp.dot(x_ref[...], w_ref[...], preferred_element_type=jnp.float32)
    hp_ref[...] = pre * dinv


def _tc_mid_body(s0_ref, s1_ref, hp_ref, dinv_ref, b_ref, g_ref, be_ref,
                 w_ref, hn_ref, hnp_ref):
    dinv = dinv_ref[...]
    z = dinv * (s0_ref[...] + s1_ref[...] + hp_ref[...]) + b_ref[...]
    mu = jnp.mean(z, axis=0, keepdims=True)
    var = jnp.mean((z - mu) ** 2, axis=0, keepdims=True)
    hn = jnp.maximum((z - mu) * lax.rsqrt(var + _EPS) * g_ref[...] + be_ref[...],
                     0.0)
    hn_ref[...] = hn
    hnp_ref[...] = jnp.dot(hn, w_ref[...],
                           preferred_element_type=jnp.float32) * dinv


def _tc_final_body(s0_ref, s1_ref, hp_ref, dinv_ref, b_ref, g_ref, be_ref,
                   res_ref, wfc_ref, bfc_ref, out_ref):
    dinv = dinv_ref[...]
    z = dinv * (s0_ref[...] + s1_ref[...] + hp_ref[...]) + b_ref[...]
    mu = jnp.mean(z, axis=0, keepdims=True)
    var = jnp.mean((z - mu) ** 2, axis=0, keepdims=True)
    h = jnp.maximum((z - mu) * lax.rsqrt(var + _EPS) * g_ref[...] + be_ref[...],
                    0.0)
    h = h + res_ref[...]
    logits = jnp.dot(h, wfc_ref[...], preferred_element_type=jnp.float32)
    out_ref[...] = jax.nn.sigmoid(logits + bfc_ref[...]) * 10.0


def _tc_call(body, out_shapes, *args):
    return pl.pallas_call(
        body,
        out_shape=out_shapes,
    )(*args)


def kernel(x, edge_index, W1, b1, g1, be1, W2, b2, g2, be2, W3, b3, g3, be3,
           Wfc, bfc):
    N, D = x.shape
    H = W1.shape[1]
    E = edge_index.shape[1]

    src = edge_index[0]
    dst = edge_index[1]

    zerosNH = jnp.zeros((N, H), jnp.float32)
    zerosND = jnp.zeros((N, _DEGW), jnp.float32)
    ones_rows = jnp.ones((_CH, _DEGW), jnp.float32)

    b1r, g1r, be1r = b1.reshape(1, H), g1.reshape(1, H), be1.reshape(1, H)
    b2r, g2r, be2r = b2.reshape(1, H), g2.reshape(1, H), be2.reshape(1, H)
    b3r, g3r, be3r = b3.reshape(1, H), g3.reshape(1, H), be3.reshape(1, H)
    bfcr = bfc.reshape(1, 1)

    deg_k = _make_deg_kernel(N, E)
    scat_k = _make_scatter_kernel(N, E, H)

    degp = deg_k(dst, ones_rows, zerosND)

    h1p, dinv = _tc_call(
        _tc0_body,
        (jax.ShapeDtypeStruct((N, H), jnp.float32),
         jax.ShapeDtypeStruct((N, 1), jnp.float32)),
        x, W1, degp[0], degp[1])

    S1 = scat_k(h1p, src, dst, zerosNH)
    h1, h2p = _tc_call(
        _tc_mid_body,
        (jax.ShapeDtypeStruct((N, H), jnp.float32),
         jax.ShapeDtypeStruct((N, H), jnp.float32)),
        S1[0], S1[1], h1p, dinv, b1r, g1r, be1r, W2)

    S2 = scat_k(h2p, src, dst, zerosNH)
    _, h3p = _tc_call(
        _tc_mid_body,
        (jax.ShapeDtypeStruct((N, H), jnp.float32),
         jax.ShapeDtypeStruct((N, H), jnp.float32)),
        S2[0], S2[1], h2p, dinv, b2r, g2r, be2r, W3)

    S3 = scat_k(h3p, src, dst, zerosNH)
    out = _tc_call(
        _tc_final_body,
        jax.ShapeDtypeStruct((N, 1), jnp.float32),
        S3[0], S3[1], h3p, dinv, b3r, g3r, be3r, h1, Wfc, bfcr)

    return out
